# baseline (speedup 1.0000x reference)
import functools

import jax
import jax.numpy as jnp
from jax import lax
from jax.experimental import pallas as pl
from jax.experimental.pallas import tpu as pltpu
from jax.experimental.pallas import tpu_sc as plsc

N_NODES = 10000
N_EDGES = 320000
D_FEAT = 128
N_HEADS = 8
D_HEAD = 16

NC = 2
NS = 16
NW = NC * NS
EPT = N_EDGES // NW
BLK = 80
NG = BLK // 16
NBLK = EPT // BLK
RPT = 624
TAIL = N_NODES - NS * RPT


def _proj_body(x_ref, bq_ref, bkv_ref, q_ref, kv_ref):
    xb = x_ref[...]
    q_ref[...] = jnp.dot(xb, bq_ref[...], preferred_element_type=jnp.float32)
    kv_ref[...] = jnp.dot(xb, bkv_ref[...], preferred_element_type=jnp.float32)


def _project(x, bq, bkv):
    nb = 10
    rows = N_NODES // nb
    return pl.pallas_call(
        _proj_body,
        grid=(nb,),
        in_specs=[
            pl.BlockSpec((rows, D_FEAT), lambda i: (i, 0)),
            pl.BlockSpec((D_FEAT, D_FEAT), lambda i: (0, 0)),
            pl.BlockSpec((D_FEAT, 2 * D_FEAT), lambda i: (0, 0)),
        ],
        out_specs=[
            pl.BlockSpec((rows, D_FEAT), lambda i: (i, 0)),
            pl.BlockSpec((rows, 2 * D_FEAT), lambda i: (i, 0)),
        ],
        out_shape=[
            jax.ShapeDtypeStruct((N_NODES, D_FEAT), jnp.float32),
            jax.ShapeDtypeStruct((N_NODES, 2 * D_FEAT), jnp.float32),
        ],
    )(x, bq, bkv)


def _scale_body(p_ref, m_ref, o_ref):
    o_ref[...] = p_ref[...] * m_ref[...] * 0.25


def _edge_scale(phi, msk):
    rows = N_EDGES // 256
    out = pl.pallas_call(
        _scale_body,
        grid=(1,),
        in_specs=[
            pl.BlockSpec((rows, 256), lambda i: (0, 0)),
            pl.BlockSpec((rows, 256), lambda i: (0, 0)),
        ],
        out_specs=pl.BlockSpec((rows, 256), lambda i: (0, 0)),
        out_shape=jax.ShapeDtypeStruct((rows, 256), jnp.float32),
    )(phi.reshape(rows, 256), msk.reshape(rows, 256))
    return out.reshape(N_EDGES)


def _combine_body(p_ref, o_ref):
    o_ref[...] = p_ref[0] + p_ref[1]


def _combine(partials):
    nb = 10
    rows = N_NODES // nb
    return pl.pallas_call(
        _combine_body,
        grid=(nb,),
        in_specs=[pl.BlockSpec((2, rows, D_FEAT), lambda i: (0, i, 0))],
        out_specs=pl.BlockSpec((rows, D_FEAT), lambda i: (i, 0)),
        out_shape=jax.ShapeDtypeStruct((N_NODES, D_FEAT), jnp.float32),
    )(partials)


_MESH = plsc.VectorSubcoreMesh(core_axis_name="c", subcore_axis_name="s")


@functools.partial(
    pl.kernel,
    out_type=jax.ShapeDtypeStruct((NC, N_NODES, D_FEAT), jnp.float32),
    mesh=_MESH,
    scratch_types=[
        pltpu.VMEM((NG, 16), jnp.int32),
        pltpu.VMEM((NG, 16), jnp.int32),
        pltpu.VMEM((BLK,), jnp.float32),
        pltpu.VMEM((BLK, D_FEAT), jnp.float32),
        pltpu.VMEM((BLK, D_FEAT), jnp.float32),
        pltpu.VMEM((BLK, 2 * D_FEAT), jnp.float32),
        pltpu.VMEM((2, 16, D_FEAT), jnp.float32),
        pltpu.VMEM_SHARED((N_NODES, D_FEAT), jnp.float32),
        pltpu.SemaphoreType.DMA,
        pltpu.SemaphoreType.DMA,
        pltpu.SemaphoreType.DMA,
        pltpu.SemaphoreType.DMA,
        pltpu.SemaphoreType.DMA,
        pltpu.SemaphoreType.DMA,
        pltpu.SemaphoreType.DMA,
        pltpu.SemaphoreType.DMA,
        pltpu.SemaphoreType.DMA,
    ],
    compiler_params=pltpu.CompilerParams(needs_layout_passes=False),
)
def _sc_edges(q_hbm, kv_hbm, w_hbm, sc_hbm, ii3_hbm, ij3_hbm,
              zero_hbm, out_hbm,
              ii_g, ij_g, sc_v, w_v, q_v, kv_v, msg_v, part,
              semL, semQ0, semQ1, semK0, semK1, semW0, semW1, semS0, semS1):
    cid = lax.axis_index("c")
    sid = lax.axis_index("s")
    wid = cid * NS + sid
    semS = (semS0, semS1)
    semQ = (semQ0, semQ1)
    semK = (semK0, semK1)
    semW = (semW0, semW1)

    pltpu.sync_copy(zero_hbm.at[pl.ds(sid * RPT, RPT)],
                    part.at[pl.ds(sid * RPT, RPT)])

    @pl.when(sid == 0)
    def _zero_tail():
        pltpu.sync_copy(zero_hbm.at[pl.ds(NS * RPT, TAIL)],
                        part.at[pl.ds(NS * RPT, TAIL)])

    plsc.subcore_barrier()

    def block(b, carry):
        base = wid * EPT + b * BLK
        blkid = wid * NBLK + b
        lin = [
            pltpu.async_copy(ii3_hbm.at[blkid], ii_g, semL),
            pltpu.async_copy(ij3_hbm.at[blkid], ij_g, semL),
            pltpu.async_copy(sc_hbm.at[pl.ds(base, BLK)], sc_v, semL),
        ]
        for cp in lin:
            cp.wait()

        def issue(g):
            s = g % 2
            rows = pl.ds(g * 16, 16)
            return (
                pltpu.async_copy(q_hbm.at[ii_g.at[g]], q_v.at[rows], semQ[s]),
                pltpu.async_copy(kv_hbm.at[ij_g.at[g]], kv_v.at[rows],
                                 semK[s]),
                pltpu.async_copy(w_hbm.at[pl.ds(base + g * 16, 16)],
                                 w_v.at[rows], semW[s]),
            )

        iota = lax.iota(jnp.int32, 16)
        pend = {0: issue(0)}
        scat = {}
        for g in range(NG):
            s = g % 2
            if g + 1 < NG:
                pend[g + 1] = issue(g + 1)
            for cp in pend.pop(g):
                cp.wait()
            if g >= 2:
                scat[g - 2].wait()
            sv = sc_v[pl.ds(g * 16, 16)]

            @plsc.parallel_loop(0, 16, 1, unroll=4)
            def _edge(j, g=g, s=s, sv=sv):
                e = g * 16 + j
                scv = jnp.take(sv, iota * 0 + j)
                for h in range(N_HEADS):
                    qh = q_v[e, pl.ds(h * D_HEAD, D_HEAD)]
                    wh = w_v[e, pl.ds(h * D_HEAD, D_HEAD)]
                    kh = kv_v[e, pl.ds(h * D_HEAD, D_HEAD)]
                    vh = kv_v[e, pl.ds(D_FEAT + h * D_HEAD, D_HEAD)]
                    p = qh * wh * kh
                    for sh in (8, 4, 2, 1):
                        p = p + jnp.take(p, (iota + sh) & 15)
                    msg_v[s, j, pl.ds(h * D_HEAD, D_HEAD)] = vh * (p * scv)

            scat[g] = pltpu.async_copy(msg_v.at[s], part.at[ii_g.at[g]],
                                       semS[s], add=True)
        scat[NG - 2].wait()
        scat[NG - 1].wait()
        return carry

    lax.fori_loop(0, NBLK, block, 0)
    plsc.subcore_barrier()
    pltpu.sync_copy(part.at[pl.ds(sid * RPT, RPT)],
                    out_hbm.at[cid, pl.ds(sid * RPT, RPT)])

    @pl.when(sid == 0)
    def _write_tail():
        pltpu.sync_copy(part.at[pl.ds(NS * RPT, TAIL)],
                        out_hbm.at[cid, pl.ds(NS * RPT, TAIL)])


def kernel(x, w_ij, phi_r_cut, idx_i, idx_j, pair_mask, Wq, Wk, Wv):
    eye = jnp.eye(N_HEADS, dtype=jnp.float32)
    bq = jnp.einsum('hfg,hk->hfkg', Wq, eye).reshape(D_FEAT, D_FEAT)
    bk = jnp.einsum('hfg,hk->hfkg', Wk, eye).reshape(D_FEAT, D_FEAT)
    bv = jnp.einsum('hfg,hk->hfkg', Wv, eye).reshape(D_FEAT, D_FEAT)
    bkv = jnp.concatenate([bk, bv], axis=1)

    q, kv = _project(x, bq, bkv)
    scale = _edge_scale(phi_r_cut, pair_mask)
    zeros = jnp.zeros((N_NODES, D_FEAT), jnp.float32)
    ii3 = idx_i.astype(jnp.int32).reshape(NW * NBLK, NG, 16)
    ij3 = idx_j.astype(jnp.int32).reshape(NW * NBLK, NG, 16)
    partials = _sc_edges(q, kv, w_ij, scale, ii3, ij3, zeros)
    return _combine(partials)


# speedup vs baseline: 1.1832x; 1.1832x over previous
import functools

import jax
import jax.numpy as jnp
import numpy as np
from jax import lax
from jax.experimental import pallas as pl
from jax.experimental.pallas import tpu as pltpu
from jax.experimental.pallas import tpu_sc as plsc

N_NODES = 10000
N_EDGES = 320000
D_FEAT = 128
N_HEADS = 8
D_HEAD = 16

NC = 2
NS = 16
NW = NC * NS
EPT = N_EDGES // NW
BLK = 80
NG = BLK // 16
NBLK = EPT // BLK
RPT = 624
TAIL = N_NODES - NS * RPT

_PERM = np.empty((D_FEAT,), np.int32)
for _c in range(D_FEAT // 32):
    for _i in range(16):
        _PERM[32 * _c + 2 * _i] = 32 * _c + _i
        _PERM[32 * _c + 2 * _i + 1] = 32 * _c + 16 + _i


def _proj_body(x_ref, bq_ref, bkv_ref, q_ref, kv_ref):
    xb = x_ref[...]
    q_ref[...] = jnp.dot(xb, bq_ref[...], preferred_element_type=jnp.float32)
    kv_ref[...] = jnp.dot(
        xb, bkv_ref[...], preferred_element_type=jnp.float32
    ).astype(jnp.bfloat16)


def _project(x, bq, bkv):
    nb = 10
    rows = N_NODES // nb
    return pl.pallas_call(
        _proj_body,
        grid=(nb,),
        in_specs=[
            pl.BlockSpec((rows, D_FEAT), lambda i: (i, 0)),
            pl.BlockSpec((D_FEAT, D_FEAT), lambda i: (0, 0)),
            pl.BlockSpec((D_FEAT, 2 * D_FEAT), lambda i: (0, 0)),
        ],
        out_specs=[
            pl.BlockSpec((rows, D_FEAT), lambda i: (i, 0)),
            pl.BlockSpec((rows, 2 * D_FEAT), lambda i: (i, 0)),
        ],
        out_shape=[
            jax.ShapeDtypeStruct((N_NODES, D_FEAT), jnp.float32),
            jax.ShapeDtypeStruct((N_NODES, 2 * D_FEAT), jnp.bfloat16),
        ],
    )(x, bq, bkv)


def _scale_body(p_ref, m_ref, o_ref):
    o_ref[...] = p_ref[...] * m_ref[...] * 0.25


def _edge_scale(phi, msk):
    rows = N_EDGES // 256
    out = pl.pallas_call(
        _scale_body,
        grid=(1,),
        in_specs=[
            pl.BlockSpec((rows, 256), lambda i: (0, 0)),
            pl.BlockSpec((rows, 256), lambda i: (0, 0)),
        ],
        out_specs=pl.BlockSpec((rows, 256), lambda i: (0, 0)),
        out_shape=jax.ShapeDtypeStruct((rows, 256), jnp.float32),
    )(phi.reshape(rows, 256), msk.reshape(rows, 256))
    return out.reshape(N_EDGES)


def _combine_body(p_ref, o_ref):
    o_ref[...] = p_ref[0] + p_ref[1]


def _combine(partials):
    nb = 10
    rows = N_NODES // nb
    return pl.pallas_call(
        _combine_body,
        grid=(nb,),
        in_specs=[pl.BlockSpec((2, rows, D_FEAT), lambda i: (0, i, 0))],
        out_specs=pl.BlockSpec((rows, D_FEAT), lambda i: (i, 0)),
        out_shape=jax.ShapeDtypeStruct((N_NODES, D_FEAT), jnp.float32),
    )(partials)


_MESH = plsc.VectorSubcoreMesh(core_axis_name="c", subcore_axis_name="s")


@functools.partial(
    pl.kernel,
    out_type=jax.ShapeDtypeStruct((NC, N_NODES, D_FEAT), jnp.float32),
    mesh=_MESH,
    scratch_types=[
        pltpu.VMEM((BLK,), jnp.int32),
        pltpu.VMEM((BLK,), jnp.int32),
        pltpu.VMEM((BLK,), jnp.float32),
        pltpu.VMEM((BLK, D_FEAT), jnp.float32),
        pltpu.VMEM((BLK, D_FEAT), jnp.float32),
        pltpu.VMEM((BLK, D_FEAT), jnp.int32),
        pltpu.VMEM((BLK, D_FEAT), jnp.float32),
        pltpu.VMEM_SHARED((N_NODES, D_FEAT), jnp.float32),
        pltpu.SemaphoreType.DMA,
        pltpu.SemaphoreType.DMA,
        pltpu.SemaphoreType.DMA,
    ],
    compiler_params=pltpu.CompilerParams(needs_layout_passes=False),
)
def _sc_edges(q_hbm, kv_hbm, w_hbm, sc_hbm, ii_hbm, ij_hbm,
              zero_hbm, out_hbm,
              ii_v, ij_v, sc_v, w_v, q_v, kv_v, msg_v, part,
              semL, semQ, semK):
    cid = lax.axis_index("c")
    sid = lax.axis_index("s")
    wid = cid * NS + sid

    pltpu.sync_copy(zero_hbm.at[pl.ds(sid * RPT, RPT)],
                    part.at[pl.ds(sid * RPT, RPT)])

    @pl.when(sid == 0)
    def _zero_tail():
        pltpu.sync_copy(zero_hbm.at[pl.ds(NS * RPT, TAIL)],
                        part.at[pl.ds(NS * RPT, TAIL)])

    plsc.subcore_barrier()

    def block(b, carry):
        base = wid * EPT + b * BLK
        lin = [
            pltpu.async_copy(ii_hbm.at[pl.ds(base, BLK)], ii_v, semL),
            pltpu.async_copy(ij_hbm.at[pl.ds(base, BLK)], ij_v, semL),
            pltpu.async_copy(sc_hbm.at[pl.ds(base, BLK)], sc_v, semL),
            pltpu.async_copy(w_hbm.at[pl.ds(base, BLK)], w_v, semL),
        ]
        for cp in lin:
            cp.wait()
        gq = pltpu.async_copy(q_hbm.at[ii_v], q_v, semQ)
        gk = pltpu.async_copy(kv_hbm.at[ij_v], kv_v, semK)
        gq.wait()
        gk.wait()

        iota = lax.iota(jnp.int32, 16)

        def group(g, carry2):
            sv = sc_v[pl.ds(g * 16, 16)]

            @plsc.parallel_loop(0, 16, 1, unroll=4)
            def _edge(j):
                e = g * 16 + j
                scv = jnp.take(sv, iota * 0 + j)
                for c in range(D_FEAT // 32):
                    kk = plsc.bitcast(kv_v[e, pl.ds(16 * c, 16)],
                                      jnp.bfloat16)
                    vv = plsc.bitcast(
                        kv_v[e, pl.ds(D_FEAT // 2 + 16 * c, 16)],
                        jnp.bfloat16)
                    ka, kb = plsc.unpack(
                        kk, format=plsc.PackFormat.INTERLEAVED)
                    va, vb = plsc.unpack(
                        vv, format=plsc.PackFormat.INTERLEAVED)
                    for hh, kx, vx in ((2 * c, ka, va),
                                       (2 * c + 1, kb, vb)):
                        qx = q_v[e, pl.ds(hh * D_HEAD, D_HEAD)]
                        wh = w_v[e, pl.ds(hh * D_HEAD, D_HEAD)]
                        p = qx * wh * kx
                        for sh in (8, 4, 2, 1):
                            p = p + jnp.take(p, (iota + sh) & 15)
                        msg_v[e, pl.ds(hh * D_HEAD, D_HEAD)] = vx * (p * scv)

            return carry2

        lax.fori_loop(0, NG, group, 0)
        pltpu.sync_copy(msg_v, part.at[ii_v], add=True)
        return carry

    lax.fori_loop(0, NBLK, block, 0)
    plsc.subcore_barrier()
    pltpu.sync_copy(part.at[pl.ds(sid * RPT, RPT)],
                    out_hbm.at[cid, pl.ds(sid * RPT, RPT)])

    @pl.when(sid == 0)
    def _write_tail():
        pltpu.sync_copy(part.at[pl.ds(NS * RPT, TAIL)],
                        out_hbm.at[cid, pl.ds(NS * RPT, TAIL)])


def kernel(x, w_ij, phi_r_cut, idx_i, idx_j, pair_mask, Wq, Wk, Wv):
    eye = jnp.eye(N_HEADS, dtype=jnp.float32)
    bq = jnp.einsum('hfg,hk->hfkg', Wq, eye).reshape(D_FEAT, D_FEAT)
    bk = jnp.einsum('hfg,hk->hfkg', Wk, eye).reshape(D_FEAT, D_FEAT)
    bv = jnp.einsum('hfg,hk->hfkg', Wv, eye).reshape(D_FEAT, D_FEAT)
    perm = jnp.asarray(_PERM)
    bkv = jnp.concatenate([bk[:, perm], bv[:, perm]], axis=1)

    q, kv = _project(x, bq, bkv)
    kv = lax.bitcast_convert_type(
        kv.reshape(N_NODES, D_FEAT, 2), jnp.int32)
    scale = _edge_scale(phi_r_cut, pair_mask)
    zeros = jnp.zeros((N_NODES, D_FEAT), jnp.float32)
    partials = _sc_edges(q, kv, w_ij, scale,
                         idx_i.astype(jnp.int32), idx_j.astype(jnp.int32),
                         zeros)
    return _combine(partials)
